# baseline (speedup 1.0000x reference)
import jax
import jax.numpy as jnp
from jax.experimental import pallas as pl
from jax.experimental.pallas import tpu as pltpu

B, S, H = 1, 2048, 1024
F = 3584
E = 8
R = 32

TT = 1024
FT = 512
NJ = F // FT
NI = S // TT

_NT = (((1,), (1,)), ((), ()))


def _moe_kernel(hs_ref, gate_ref, w1_ref, w3_ref, w2_ref,
                w1a_ref, w3a_ref, w1b_ref, w3b_ref, w2a_ref, w2b_ref,
                final_ref, logits_ref,
                hsb_s, we_s, a1_s, a3_s, p2_s):
    j = pl.program_id(1)
    f32 = jnp.float32

    @pl.when(j == 0)
    def _prologue():
        hsf = hs_ref[:]
        hsb = hsf.astype(jnp.bfloat16)
        hsb_s[:] = hsb
        logits = jax.lax.dot_general(hsf, gate_ref[:], _NT,
                                     preferred_element_type=f32)
        logits_ref[:] = logits
        p = jax.nn.softmax(logits, axis=-1)
        v1 = jnp.max(p, axis=-1)
        i1 = jnp.argmax(p, axis=-1)
        iota = jax.lax.broadcasted_iota(jnp.int32, (TT, E), 1)
        m1 = iota == i1[:, None]
        pm = jnp.where(m1, -jnp.inf, p)
        v2 = jnp.max(pm, axis=-1)
        i2 = jnp.argmax(pm, axis=-1)
        denom = v1 + v2
        we = (jnp.where(m1, (v1 / denom)[:, None], 0.0)
              + jnp.where(iota == i2[:, None], (v2 / denom)[:, None], 0.0))
        we_s[:] = we
        a1_s[:] = jax.lax.dot_general(hsb, w1a_ref[:], _NT,
                                      preferred_element_type=f32
                                      ).astype(jnp.bfloat16)
        a3_s[:] = jax.lax.dot_general(hsb, w3a_ref[:], _NT,
                                      preferred_element_type=f32
                                      ).astype(jnp.bfloat16)
        p2_s[...] = jnp.zeros_like(p2_s)
        final_ref[:] = jnp.zeros_like(final_ref)

    hsb = hsb_s[:]
    base1 = jax.lax.dot_general(hsb, w1_ref[:], _NT,
                                preferred_element_type=f32)
    base3 = jax.lax.dot_general(hsb, w3_ref[:], _NT,
                                preferred_element_type=f32)
    cx2 = None
    for e in range(E):
        a1e = a1_s[:, e * R:(e + 1) * R]
        a3e = a3_s[:, e * R:(e + 1) * R]
        x1 = base1 + jax.lax.dot_general(a1e, w1b_ref[e], _NT,
                                         preferred_element_type=f32)
        x3 = base3 + jax.lax.dot_general(a3e, w3b_ref[e], _NT,
                                         preferred_element_type=f32)
        x2 = x1 * jax.nn.sigmoid(x1) * x3
        wx2 = we_s[:, e][:, None] * x2
        cx2 = wx2 if cx2 is None else cx2 + wx2
        wx2b = wx2.astype(jnp.bfloat16)
        pe = jax.lax.dot_general(wx2b, w2a_ref[e], _NT,
                                 preferred_element_type=f32)
        p2_s[e] += pe

    final_ref[:] += jax.lax.dot_general(cx2.astype(jnp.bfloat16), w2_ref[:],
                                        _NT, preferred_element_type=f32)

    @pl.when(j == NJ - 1)
    def _epilogue():
        acc = final_ref[:]
        for e in range(E):
            acc += jnp.dot(p2_s[e].astype(jnp.bfloat16), w2b_ref[e],
                           preferred_element_type=f32)
        final_ref[:] = acc


def kernel(hidden_states, gate_w, w1, w2, w3, w1_a, w1_b, w2_a, w2_b,
           w3_a, w3_b, interpret=False):
    hs = hidden_states.reshape(-1, H)
    bf = jnp.bfloat16
    w1a_cat = w1_a.reshape(E * R, H).astype(bf)
    w3a_cat = w3_a.reshape(E * R, H).astype(bf)
    w2b_t = jnp.transpose(w2_b, (0, 2, 1)).astype(bf)

    grid = (NI, NJ)
    final, logits = pl.pallas_call(
        _moe_kernel,
        grid=grid,
        in_specs=[
            pl.BlockSpec((TT, H), lambda i, j: (i, 0)),
            pl.BlockSpec((E, H), lambda i, j: (0, 0)),
            pl.BlockSpec((FT, H), lambda i, j: (j, 0)),
            pl.BlockSpec((FT, H), lambda i, j: (j, 0)),
            pl.BlockSpec((H, FT), lambda i, j: (0, j)),
            pl.BlockSpec((E * R, H), lambda i, j: (0, 0)),
            pl.BlockSpec((E * R, H), lambda i, j: (0, 0)),
            pl.BlockSpec((E, FT, R), lambda i, j: (0, j, 0)),
            pl.BlockSpec((E, FT, R), lambda i, j: (0, j, 0)),
            pl.BlockSpec((E, R, FT), lambda i, j: (0, 0, j)),
            pl.BlockSpec((E, R, H), lambda i, j: (0, 0, 0)),
        ],
        out_specs=[
            pl.BlockSpec((TT, H), lambda i, j: (i, 0)),
            pl.BlockSpec((TT, E), lambda i, j: (i, 0)),
        ],
        out_shape=[
            jax.ShapeDtypeStruct((S, H), jnp.float32),
            jax.ShapeDtypeStruct((S, E), jnp.float32),
        ],
        scratch_shapes=[
            pltpu.VMEM((TT, H), bf),
            pltpu.VMEM((TT, E), jnp.float32),
            pltpu.VMEM((TT, E * R), bf),
            pltpu.VMEM((TT, E * R), bf),
            pltpu.VMEM((E, TT, R), jnp.float32),
        ],
        compiler_params=pltpu.CompilerParams(
            dimension_semantics=("parallel", "arbitrary"),
        ),
        interpret=interpret,
    )(hs, gate_w, w1.astype(bf), w3.astype(bf), w2.astype(bf),
      w1a_cat, w3a_cat, w1_b.astype(bf), w3_b.astype(bf), w2_a.astype(bf),
      w2b_t)
    return final.reshape(B, S, H), logits


# speedup vs baseline: 1.1781x; 1.1781x over previous
import jax
import jax.numpy as jnp
from jax.experimental import pallas as pl
from jax.experimental.pallas import tpu as pltpu

B, S, H = 1, 2048, 1024
F = 3584
E = 8
R = 32

TT = 1024
FT = 512
NJ = F // FT
NI = S // TT

_NT = (((1,), (1,)), ((), ()))


def _moe_kernel(hs_ref, gate_ref, w1_ref, w3_ref, w2_ref,
                w1a_ref, w3a_ref, w1b_ref, w3b_ref, w2a_ref, w2b_ref,
                final_ref, logits_ref,
                hsb_s, we_s, a1_s, a3_s, p2_s):
    j = pl.program_id(1)
    f32 = jnp.float32

    @pl.when(j == 0)
    def _prologue():
        hsf = hs_ref[:]
        hsb = hsf.astype(jnp.bfloat16)
        hsb_s[:] = hsb
        logits = jax.lax.dot_general(hsf, gate_ref[:], _NT,
                                     preferred_element_type=f32)
        logits_ref[:] = logits
        p = jax.nn.softmax(logits, axis=-1)
        v1 = jnp.max(p, axis=-1)
        i1 = jnp.argmax(p, axis=-1)
        iota = jax.lax.broadcasted_iota(jnp.int32, (TT, E), 1)
        m1 = iota == i1[:, None]
        pm = jnp.where(m1, -jnp.inf, p)
        v2 = jnp.max(pm, axis=-1)
        i2 = jnp.argmax(pm, axis=-1)
        denom = v1 + v2
        we = (jnp.where(m1, (v1 / denom)[:, None], 0.0)
              + jnp.where(iota == i2[:, None], (v2 / denom)[:, None], 0.0))
        we_s[:] = we
        a1_s[:] = jax.lax.dot_general(hsb, w1a_ref[:], _NT,
                                      preferred_element_type=f32
                                      ).astype(jnp.bfloat16)
        a3_s[:] = jax.lax.dot_general(hsb, w3a_ref[:], _NT,
                                      preferred_element_type=f32
                                      ).astype(jnp.bfloat16)
        p2_s[...] = jnp.zeros_like(p2_s)
        final_ref[:] = jnp.zeros_like(final_ref)

    bf16 = jnp.bfloat16
    hsb = hsb_s[:]
    base1 = jax.lax.dot_general(hsb, w1_ref[:], _NT,
                                preferred_element_type=f32
                                ).astype(bf16)
    base3 = jax.lax.dot_general(hsb, w3_ref[:], _NT,
                                preferred_element_type=f32
                                ).astype(bf16)
    cx2 = None
    for e in range(E):
        a1e = a1_s[:, e * R:(e + 1) * R]
        a3e = a3_s[:, e * R:(e + 1) * R]
        x1 = base1 + jax.lax.dot_general(a1e, w1b_ref[e], _NT,
                                         preferred_element_type=f32
                                         ).astype(bf16)
        x3 = base3 + jax.lax.dot_general(a3e, w3b_ref[e], _NT,
                                         preferred_element_type=f32
                                         ).astype(bf16)
        x2 = x1 * jax.nn.sigmoid(x1) * x3
        wx2 = we_s[:, e][:, None].astype(bf16) * x2
        cx2 = wx2 if cx2 is None else cx2 + wx2
        pe = jax.lax.dot_general(x2, w2a_ref[e], _NT,
                                 preferred_element_type=f32)
        p2_s[e] += pe

    final_ref[:] += jax.lax.dot_general(cx2, w2_ref[:],
                                        _NT, preferred_element_type=f32)

    @pl.when(j == NJ - 1)
    def _epilogue():
        acc = final_ref[:]
        for e in range(E):
            p2e = (we_s[:, e][:, None] * p2_s[e]).astype(jnp.bfloat16)
            acc += jnp.dot(p2e, w2b_ref[e], preferred_element_type=f32)
        final_ref[:] = acc


def kernel(hidden_states, gate_w, w1, w2, w3, w1_a, w1_b, w2_a, w2_b,
           w3_a, w3_b, interpret=False):
    hs = hidden_states.reshape(-1, H)
    bf = jnp.bfloat16
    w1a_cat = w1_a.reshape(E * R, H).astype(bf)
    w3a_cat = w3_a.reshape(E * R, H).astype(bf)
    w2b_t = jnp.transpose(w2_b, (0, 2, 1)).astype(bf)

    grid = (NI, NJ)
    final, logits = pl.pallas_call(
        _moe_kernel,
        grid=grid,
        in_specs=[
            pl.BlockSpec((TT, H), lambda i, j: (i, 0)),
            pl.BlockSpec((E, H), lambda i, j: (0, 0)),
            pl.BlockSpec((FT, H), lambda i, j: (j, 0)),
            pl.BlockSpec((FT, H), lambda i, j: (j, 0)),
            pl.BlockSpec((H, FT), lambda i, j: (0, j)),
            pl.BlockSpec((E * R, H), lambda i, j: (0, 0)),
            pl.BlockSpec((E * R, H), lambda i, j: (0, 0)),
            pl.BlockSpec((E, FT, R), lambda i, j: (0, j, 0)),
            pl.BlockSpec((E, FT, R), lambda i, j: (0, j, 0)),
            pl.BlockSpec((E, R, FT), lambda i, j: (0, 0, j)),
            pl.BlockSpec((E, R, H), lambda i, j: (0, 0, 0)),
        ],
        out_specs=[
            pl.BlockSpec((TT, H), lambda i, j: (i, 0)),
            pl.BlockSpec((TT, E), lambda i, j: (i, 0)),
        ],
        out_shape=[
            jax.ShapeDtypeStruct((S, H), jnp.float32),
            jax.ShapeDtypeStruct((S, E), jnp.float32),
        ],
        scratch_shapes=[
            pltpu.VMEM((TT, H), bf),
            pltpu.VMEM((TT, E), jnp.float32),
            pltpu.VMEM((TT, E * R), bf),
            pltpu.VMEM((TT, E * R), bf),
            pltpu.VMEM((E, TT, R), jnp.float32),
        ],
        compiler_params=pltpu.CompilerParams(
            dimension_semantics=("parallel", "arbitrary"),
        ),
        interpret=interpret,
    )(hs, gate_w, w1.astype(bf), w3.astype(bf), w2.astype(bf),
      w1a_cat, w3a_cat, w1_b.astype(bf), w3_b.astype(bf), w2_a.astype(bf),
      w2b_t)
    return final.reshape(B, S, H), logits


# speedup vs baseline: 1.2035x; 1.0215x over previous
import jax
import jax.numpy as jnp
from jax.experimental import pallas as pl
from jax.experimental.pallas import tpu as pltpu

B, S, H = 1, 2048, 1024
F = 3584
E = 8
R = 32

TT = 1024
FT = 512
NJ = F // FT
NI = S // TT

_NT = (((1,), (1,)), ((), ()))


def _moe_kernel(hs_ref, gate_ref, w1_ref, w3_ref, w2_ref,
                w1a_ref, w3a_ref, w1b_ref, w3b_ref, w2a_ref, w2b_ref,
                final_ref, logits_ref,
                hsb_s, we_s, a1_s, a3_s, p2_s):
    j = pl.program_id(1)
    f32 = jnp.float32

    @pl.when(j == 0)
    def _prologue():
        hsf = hs_ref[:]
        hsb = hsf.astype(jnp.bfloat16)
        hsb_s[:] = hsb
        logits = jax.lax.dot_general(hsf, gate_ref[:], _NT,
                                     preferred_element_type=f32)
        logits_ref[:] = logits
        p = jax.nn.softmax(logits, axis=-1)
        v1 = jnp.max(p, axis=-1)
        i1 = jnp.argmax(p, axis=-1)
        iota = jax.lax.broadcasted_iota(jnp.int32, (TT, E), 1)
        m1 = iota == i1[:, None]
        pm = jnp.where(m1, -jnp.inf, p)
        v2 = jnp.max(pm, axis=-1)
        i2 = jnp.argmax(pm, axis=-1)
        denom = v1 + v2
        we = (jnp.where(m1, (v1 / denom)[:, None], 0.0)
              + jnp.where(iota == i2[:, None], (v2 / denom)[:, None], 0.0))
        we_s[:] = we
        a1_s[:] = jax.lax.dot_general(hsb, w1a_ref[:], _NT,
                                      preferred_element_type=f32
                                      ).astype(jnp.bfloat16)
        a3_s[:] = jax.lax.dot_general(hsb, w3a_ref[:], _NT,
                                      preferred_element_type=f32
                                      ).astype(jnp.bfloat16)
        p2_s[...] = jnp.zeros_like(p2_s)
        final_ref[:] = jnp.zeros_like(final_ref)

    bf16 = jnp.bfloat16
    hsb = hsb_s[:]
    base1 = jax.lax.dot_general(hsb, w1_ref[:], _NT,
                                preferred_element_type=f32
                                ).astype(bf16)
    base3 = jax.lax.dot_general(hsb, w3_ref[:], _NT,
                                preferred_element_type=f32
                                ).astype(bf16)
    cx2 = None
    for e in range(E):
        a1e = a1_s[:, e * R:(e + 1) * R]
        a3e = a3_s[:, e * R:(e + 1) * R]
        x1 = base1 + jax.lax.dot_general(a1e, w1b_ref[e], _NT,
                                         preferred_element_type=f32
                                         ).astype(bf16)
        x3 = base3 + jax.lax.dot_general(a3e, w3b_ref[e], _NT,
                                         preferred_element_type=f32
                                         ).astype(bf16)
        x2 = x1 * jax.nn.sigmoid(x1) * x3
        wx2 = we_s[:, e][:, None].astype(bf16) * x2
        cx2 = wx2 if cx2 is None else cx2 + wx2
        pe = jax.lax.dot_general(x2, w2a_ref[e], _NT,
                                 preferred_element_type=f32)
        p2_s[e] += pe

    final_ref[:] += jax.lax.dot_general(cx2, w2_ref[:],
                                        _NT, preferred_element_type=f32)

    @pl.when(j == NJ - 1)
    def _epilogue():
        p2cat = jnp.concatenate(
            [(we_s[:, e][:, None] * p2_s[e]).astype(jnp.bfloat16)
             for e in range(E)], axis=1)
        final_ref[:] += jnp.dot(p2cat, w2b_ref[:],
                                preferred_element_type=f32)


def kernel(hidden_states, gate_w, w1, w2, w3, w1_a, w1_b, w2_a, w2_b,
           w3_a, w3_b, interpret=False):
    hs = hidden_states.reshape(-1, H)
    bf = jnp.bfloat16
    w1a_cat = w1_a.reshape(E * R, H).astype(bf)
    w3a_cat = w3_a.reshape(E * R, H).astype(bf)
    w2b_t = jnp.transpose(w2_b, (0, 2, 1)).reshape(E * R, H).astype(bf)

    grid = (NI, NJ)
    final, logits = pl.pallas_call(
        _moe_kernel,
        grid=grid,
        in_specs=[
            pl.BlockSpec((TT, H), lambda i, j: (i, 0)),
            pl.BlockSpec((E, H), lambda i, j: (0, 0)),
            pl.BlockSpec((FT, H), lambda i, j: (j, 0)),
            pl.BlockSpec((FT, H), lambda i, j: (j, 0)),
            pl.BlockSpec((H, FT), lambda i, j: (0, j)),
            pl.BlockSpec((E * R, H), lambda i, j: (0, 0)),
            pl.BlockSpec((E * R, H), lambda i, j: (0, 0)),
            pl.BlockSpec((E, FT, R), lambda i, j: (0, j, 0)),
            pl.BlockSpec((E, FT, R), lambda i, j: (0, j, 0)),
            pl.BlockSpec((E, R, FT), lambda i, j: (0, 0, j)),
            pl.BlockSpec((E * R, H), lambda i, j: (0, 0)),
        ],
        out_specs=[
            pl.BlockSpec((TT, H), lambda i, j: (i, 0)),
            pl.BlockSpec((TT, E), lambda i, j: (i, 0)),
        ],
        out_shape=[
            jax.ShapeDtypeStruct((S, H), jnp.float32),
            jax.ShapeDtypeStruct((S, E), jnp.float32),
        ],
        scratch_shapes=[
            pltpu.VMEM((TT, H), bf),
            pltpu.VMEM((TT, E), jnp.float32),
            pltpu.VMEM((TT, E * R), bf),
            pltpu.VMEM((TT, E * R), bf),
            pltpu.VMEM((E, TT, R), jnp.float32),
        ],
        compiler_params=pltpu.CompilerParams(
            dimension_semantics=("parallel", "arbitrary"),
        ),
        interpret=interpret,
    )(hs, gate_w, w1.astype(bf), w3.astype(bf), w2.astype(bf),
      w1a_cat, w3a_cat, w1_b.astype(bf), w3_b.astype(bf), w2_a.astype(bf),
      w2b_t)
    return final.reshape(B, S, H), logits


# speedup vs baseline: 1.2579x; 1.0452x over previous
import jax
import jax.numpy as jnp
from jax.experimental import pallas as pl
from jax.experimental.pallas import tpu as pltpu

B, S, H = 1, 2048, 1024
F = 3584
E = 8
R = 32

TT = S
FT = 512
NJ = F // FT

_NT = (((1,), (1,)), ((), ()))


def _router_kernel(hs_ref, gate_ref, logits_ref, we_ref):
    f32 = jnp.float32
    logits = jax.lax.dot_general(hs_ref[:], gate_ref[:], _NT,
                                 preferred_element_type=f32)
    logits_ref[:] = logits
    p = jax.nn.softmax(logits, axis=-1)
    v1 = jnp.max(p, axis=-1)
    i1 = jnp.argmax(p, axis=-1)
    iota = jax.lax.broadcasted_iota(jnp.int32, (TT, E), 1)
    m1 = iota == i1[:, None]
    pm = jnp.where(m1, -jnp.inf, p)
    v2 = jnp.max(pm, axis=-1)
    i2 = jnp.argmax(pm, axis=-1)
    denom = v1 + v2
    we_ref[:] = (jnp.where(m1, (v1 / denom)[:, None], 0.0)
                 + jnp.where(iota == i2[:, None], (v2 / denom)[:, None], 0.0))


def _moe_kernel(hs_ref, we_ref, w1_ref, w3_ref, w2_ref,
                w1a_ref, w3a_ref, w1b_ref, w3b_ref, w2a_ref, w2b_ref,
                final_ref,
                a1_s, a3_s, p2_s):
    j = pl.program_id(0)
    f32 = jnp.float32
    bf16 = jnp.bfloat16
    hsb = hs_ref[:]

    @pl.when(j == 0)
    def _prologue():
        a1_s[:] = jax.lax.dot_general(hsb, w1a_ref[:], _NT,
                                      preferred_element_type=f32
                                      ).astype(bf16)
        a3_s[:] = jax.lax.dot_general(hsb, w3a_ref[:], _NT,
                                      preferred_element_type=f32
                                      ).astype(bf16)
        p2_s[...] = jnp.zeros_like(p2_s)
        final_ref[:] = jnp.zeros_like(final_ref)

    base1 = jax.lax.dot_general(hsb, w1_ref[:], _NT,
                                preferred_element_type=f32
                                ).astype(bf16)
    base3 = jax.lax.dot_general(hsb, w3_ref[:], _NT,
                                preferred_element_type=f32
                                ).astype(bf16)
    cx2 = None
    for e in range(E):
        a1e = a1_s[:, e * R:(e + 1) * R]
        a3e = a3_s[:, e * R:(e + 1) * R]
        x1 = base1 + jax.lax.dot_general(a1e, w1b_ref[e], _NT,
                                         preferred_element_type=f32
                                         ).astype(bf16)
        x3 = base3 + jax.lax.dot_general(a3e, w3b_ref[e], _NT,
                                         preferred_element_type=f32
                                         ).astype(bf16)
        x2 = x1 * jax.nn.sigmoid(x1) * x3
        wx2 = we_ref[:, e][:, None].astype(bf16) * x2
        cx2 = wx2 if cx2 is None else cx2 + wx2
        pe = jax.lax.dot_general(x2, w2a_ref[e], _NT,
                                 preferred_element_type=f32)
        p2_s[:, e * R:(e + 1) * R] += pe

    final_ref[:] += jax.lax.dot_general(cx2, w2_ref[:],
                                        _NT, preferred_element_type=f32)

    @pl.when(j == NJ - 1)
    def _epilogue():
        wrep = jnp.concatenate(
            [jnp.broadcast_to(we_ref[:, e][:, None], (TT, R))
             for e in range(E)], axis=1)
        p2cat = (wrep * p2_s[:]).astype(bf16)
        final_ref[:] += jnp.dot(p2cat, w2b_ref[:],
                                preferred_element_type=f32)


def kernel(hidden_states, gate_w, w1, w2, w3, w1_a, w1_b, w2_a, w2_b,
           w3_a, w3_b, interpret=False):
    hs = hidden_states.reshape(-1, H)
    bf = jnp.bfloat16
    w1a_cat = w1_a.reshape(E * R, H).astype(bf)
    w3a_cat = w3_a.reshape(E * R, H).astype(bf)
    w2b_t = jnp.transpose(w2_b, (0, 2, 1)).reshape(E * R, H).astype(bf)

    logits, we = pl.pallas_call(
        _router_kernel,
        grid=(1,),
        in_specs=[
            pl.BlockSpec((TT, H), lambda j: (0, 0)),
            pl.BlockSpec((E, H), lambda j: (0, 0)),
        ],
        out_specs=[
            pl.BlockSpec((TT, E), lambda j: (0, 0)),
            pl.BlockSpec((TT, E), lambda j: (0, 0)),
        ],
        out_shape=[
            jax.ShapeDtypeStruct((S, E), jnp.float32),
            jax.ShapeDtypeStruct((S, E), jnp.float32),
        ],
        interpret=interpret,
    )(hs, gate_w)

    final, = pl.pallas_call(
        _moe_kernel,
        grid=(NJ,),
        in_specs=[
            pl.BlockSpec((TT, H), lambda j: (0, 0)),
            pl.BlockSpec((TT, E), lambda j: (0, 0)),
            pl.BlockSpec((FT, H), lambda j: (j, 0)),
            pl.BlockSpec((FT, H), lambda j: (j, 0)),
            pl.BlockSpec((H, FT), lambda j: (0, j)),
            pl.BlockSpec((E * R, H), lambda j: (0, 0)),
            pl.BlockSpec((E * R, H), lambda j: (0, 0)),
            pl.BlockSpec((E, FT, R), lambda j: (0, j, 0)),
            pl.BlockSpec((E, FT, R), lambda j: (0, j, 0)),
            pl.BlockSpec((E, R, FT), lambda j: (0, 0, j)),
            pl.BlockSpec((E * R, H), lambda j: (0, 0)),
        ],
        out_specs=[
            pl.BlockSpec((TT, H), lambda j: (0, 0)),
        ],
        out_shape=[
            jax.ShapeDtypeStruct((S, H), jnp.float32),
        ],
        scratch_shapes=[
            pltpu.VMEM((TT, E * R), bf),
            pltpu.VMEM((TT, E * R), bf),
            pltpu.VMEM((TT, E * R), jnp.float32),
        ],
        compiler_params=pltpu.CompilerParams(
            dimension_semantics=("arbitrary",),
        ),
        interpret=interpret,
    )(hs.astype(bf), we, w1.astype(bf), w3.astype(bf), w2.astype(bf),
      w1a_cat, w3a_cat, w1_b.astype(bf), w3_b.astype(bf), w2_a.astype(bf),
      w2b_t)
    return final.reshape(B, S, H), logits


# speedup vs baseline: 1.2590x; 1.0009x over previous
import jax
import jax.numpy as jnp
from jax.experimental import pallas as pl
from jax.experimental.pallas import tpu as pltpu

B, S, H = 1, 2048, 1024
F = 3584
E = 8
R = 32

TT = S
FT = 512
NJ = F // FT

_NT = (((1,), (1,)), ((), ()))


def _router_kernel(hs_ref, gate_ref, logits_ref, we_ref):
    f32 = jnp.float32
    logits = jax.lax.dot_general(hs_ref[:], gate_ref[:], _NT,
                                 preferred_element_type=f32)
    logits_ref[:] = logits
    p = jax.nn.softmax(logits, axis=-1)
    v1 = jnp.max(p, axis=-1)
    i1 = jnp.argmax(p, axis=-1)
    iota = jax.lax.broadcasted_iota(jnp.int32, (TT, E), 1)
    m1 = iota == i1[:, None]
    pm = jnp.where(m1, -jnp.inf, p)
    v2 = jnp.max(pm, axis=-1)
    i2 = jnp.argmax(pm, axis=-1)
    denom = v1 + v2
    we_ref[:] = (jnp.where(m1, (v1 / denom)[:, None], 0.0)
                 + jnp.where(iota == i2[:, None], (v2 / denom)[:, None], 0.0))


def _moe_kernel(hs_ref, we_ref, w1_ref, w3_ref, w2_ref,
                w1a_ref, w3a_ref, w1b_ref, w3b_ref, w2a_ref, w2b_ref,
                final_ref,
                a1_s, a3_s, p2_s):
    j = pl.program_id(0)
    f32 = jnp.float32
    bf16 = jnp.bfloat16
    hsb = hs_ref[:]

    @pl.when(j == 0)
    def _prologue():
        a1_s[:] = jax.lax.dot_general(hsb, w1a_ref[:], _NT,
                                      preferred_element_type=f32
                                      ).astype(bf16)
        a3_s[:] = jax.lax.dot_general(hsb, w3a_ref[:], _NT,
                                      preferred_element_type=f32
                                      ).astype(bf16)
        p2_s[...] = jnp.zeros_like(p2_s)
        final_ref[:] = jnp.zeros_like(final_ref)

    base1 = jax.lax.dot_general(hsb, w1_ref[:], _NT,
                                preferred_element_type=f32
                                ).astype(bf16)
    base3 = jax.lax.dot_general(hsb, w3_ref[:], _NT,
                                preferred_element_type=f32
                                ).astype(bf16)
    cx2 = None
    for e in range(E):
        a1e = a1_s[:, e * R:(e + 1) * R]
        a3e = a3_s[:, e * R:(e + 1) * R]
        x1 = base1 + jax.lax.dot_general(a1e, w1b_ref[e], _NT,
                                         preferred_element_type=f32
                                         ).astype(bf16)
        x3 = base3 + jax.lax.dot_general(a3e, w3b_ref[e], _NT,
                                         preferred_element_type=f32
                                         ).astype(bf16)
        x2 = x1 * jax.nn.sigmoid(x1) * x3
        wx2 = we_ref[:, e][:, None].astype(bf16) * x2
        cx2 = wx2 if cx2 is None else cx2 + wx2
        pe = jax.lax.dot_general(x2, w2a_ref[e], _NT,
                                 preferred_element_type=f32)
        p2_s[:, e * R:(e + 1) * R] += pe

    final_ref[:] += jax.lax.dot_general(cx2, w2_ref[:],
                                        _NT, preferred_element_type=f32)

    @pl.when(j == NJ - 1)
    def _epilogue():
        wrep = jnp.concatenate(
            [jnp.broadcast_to(we_ref[:, e][:, None], (TT, R))
             for e in range(E)], axis=1)
        p2cat = (wrep * p2_s[:]).astype(bf16)
        final_ref[:] += jnp.dot(p2cat, w2b_ref[:],
                                preferred_element_type=f32)


def kernel(hidden_states, gate_w, w1, w2, w3, w1_a, w1_b, w2_a, w2_b,
           w3_a, w3_b):
    hs = hidden_states.reshape(-1, H)
    bf = jnp.bfloat16
    w1a_cat = w1_a.reshape(E * R, H).astype(bf)
    w3a_cat = w3_a.reshape(E * R, H).astype(bf)
    w2b_t = jnp.transpose(w2_b, (0, 2, 1)).reshape(E * R, H).astype(bf)

    logits, we = pl.pallas_call(
        _router_kernel,
        grid=(1,),
        in_specs=[
            pl.BlockSpec((TT, H), lambda j: (0, 0)),
            pl.BlockSpec((E, H), lambda j: (0, 0)),
        ],
        out_specs=[
            pl.BlockSpec((TT, E), lambda j: (0, 0)),
            pl.BlockSpec((TT, E), lambda j: (0, 0)),
        ],
        out_shape=[
            jax.ShapeDtypeStruct((S, E), jnp.float32),
            jax.ShapeDtypeStruct((S, E), jnp.float32),
        ],
    )(hs, gate_w)

    final, = pl.pallas_call(
        _moe_kernel,
        grid=(NJ,),
        in_specs=[
            pl.BlockSpec((TT, H), lambda j: (0, 0)),
            pl.BlockSpec((TT, E), lambda j: (0, 0)),
            pl.BlockSpec((FT, H), lambda j: (j, 0)),
            pl.BlockSpec((FT, H), lambda j: (j, 0)),
            pl.BlockSpec((H, FT), lambda j: (0, j)),
            pl.BlockSpec((E * R, H), lambda j: (0, 0)),
            pl.BlockSpec((E * R, H), lambda j: (0, 0)),
            pl.BlockSpec((E, FT, R), lambda j: (0, j, 0)),
            pl.BlockSpec((E, FT, R), lambda j: (0, j, 0)),
            pl.BlockSpec((E, R, FT), lambda j: (0, 0, j)),
            pl.BlockSpec((E * R, H), lambda j: (0, 0)),
        ],
        out_specs=[
            pl.BlockSpec((TT, H), lambda j: (0, 0)),
        ],
        out_shape=[
            jax.ShapeDtypeStruct((S, H), jnp.float32),
        ],
        scratch_shapes=[
            pltpu.VMEM((TT, E * R), bf),
            pltpu.VMEM((TT, E * R), bf),
            pltpu.VMEM((TT, E * R), jnp.float32),
        ],
        compiler_params=pltpu.CompilerParams(
            dimension_semantics=("arbitrary",),
        ),
    )(hs.astype(bf), we, w1.astype(bf), w3.astype(bf), w2.astype(bf),
      w1a_cat, w3a_cat, w1_b.astype(bf), w3_b.astype(bf), w2_a.astype(bf),
      w2b_t)
    return final.reshape(B, S, H), logits
